# baseline (speedup 1.0000x reference)
import functools

import jax
import jax.numpy as jnp
from jax import lax
from jax.experimental import pallas as pl
from jax.experimental.pallas import tpu as pltpu
from jax.experimental.pallas import tpu_sc as plsc

_B = 4096
_FD = 10000
_XW = 30000
_SCALE_TANH = 1.2
_BB = 128
_NW = 32
_RPW = _B // _NW
_RV = 16
_XROWS = _B * _XW // _RV
_L = 16


_CB = 2048
_NJ = -(-_FD // _CB)


def _tc_argmax_copy(x_ref, logits_ref, c_ref, m_s, i_s):
    j = pl.program_id(1)
    xb = x_ref[...]
    logits_ref[...] = xb
    col = j * _CB + lax.broadcasted_iota(jnp.int32, xb.shape, 1)
    valid = col < _FD
    xm = jnp.where(valid, xb, -jnp.inf)
    bm = jnp.max(xm, axis=1)
    cand = jnp.where(xm == bm[:, None], col, _FD - 1)
    bi = jnp.min(cand, axis=1)

    @pl.when(j == 0)
    def _():
        m_s[...] = bm
        i_s[...] = bi

    @pl.when(j > 0)
    def _():
        better = bm > m_s[...]
        m_s[...] = jnp.where(better, bm, m_s[...])
        i_s[...] = jnp.where(better, bi, i_s[...])

    @pl.when(j == _NJ - 1)
    def _():
        c_ref[...] = i_s[...]


def _sc_gather_tail(xf_hbm, c_hbm, ctrf_hbm, upf_hbm, dnf_hbm,
                    gps_hbm, inv_hbm, ctr_hbm, reg_hbm,
                    c_v, r0_v, r1_v, i20_v, i21_v,
                    x0_v, x1_v, ct0_v, ct1_v, up0_v, up1_v, dn0_v, dn1_v,
                    gps_v, inv_v, ctr_v, reg_v, sem):
    wid = lax.axis_index("s") * 2 + lax.axis_index("c")
    base = wid * _RPW

    pltpu.sync_copy(c_hbm.at[pl.ds(base, _RPW)], c_v)

    for j in range(_RPW // _L):
        sl = pl.ds(j * _L, _L)
        cj = c_v[sl]
        rows = base + j * _L + lax.iota(jnp.int32, _L)
        f = rows * _XW + _FD + 2 * cj
        r0_v[sl] = f
        r1_v[sl] = f + 1
        i20_v[sl] = 2 * cj
        i21_v[sl] = 2 * cj + 1

    cps = [
        pltpu.async_copy(xf_hbm.at[r0_v], x0_v, sem),
        pltpu.async_copy(xf_hbm.at[r1_v], x1_v, sem),
        pltpu.async_copy(ctrf_hbm.at[i20_v], ct0_v, sem),
        pltpu.async_copy(ctrf_hbm.at[i21_v], ct1_v, sem),
        pltpu.async_copy(upf_hbm.at[i20_v], up0_v, sem),
        pltpu.async_copy(upf_hbm.at[i21_v], up1_v, sem),
        pltpu.async_copy(dnf_hbm.at[i20_v], dn0_v, sem),
        pltpu.async_copy(dnf_hbm.at[i21_v], dn1_v, sem),
    ]
    for cp in cps:
        cp.wait()

    for comp in range(2):
        x_v = x0_v if comp == 0 else x1_v
        cc_v = ct0_v if comp == 0 else ct1_v
        uu_v = up0_v if comp == 0 else up1_v
        dd_v = dn0_v if comp == 0 else dn1_v
        deg = 90.0 if comp == 0 else 180.0
        for k in range(_RPW // _L):
            sl = pl.ds(k * _L, _L)
            osl = pl.ds(comp * _RPW + k * _L, _L)
            a = x_v[sl]
            absa = jnp.minimum(jnp.abs(a), 30.0)
            t = 1.0 - 2.0 / (jnp.exp(2.0 * absa) + 1.0)
            reg = _SCALE_TANH * jnp.sign(a) * t
            ctr = cc_v[sl]
            size = jnp.where(reg > 0, uu_v[sl], dd_v[sl])
            g = jnp.clip(ctr + reg * size, -1.0, 1.0) * deg
            gps_v[osl] = g
            inv_v[osl] = 1.0 / size
            ctr_v[osl] = ctr
            reg_v[osl] = reg

    for comp in range(2):
        ob = comp * _B + base
        isl = pl.ds(comp * _RPW, _RPW)
        osl = pl.ds(ob, _RPW)
        pltpu.sync_copy(gps_v.at[isl], gps_hbm.at[osl])
        pltpu.sync_copy(inv_v.at[isl], inv_hbm.at[osl])
        pltpu.sync_copy(ctr_v.at[isl], ctr_hbm.at[osl])
        pltpu.sync_copy(reg_v.at[isl], reg_hbm.at[osl])


def kernel(x, gt_label, cell_center, cell_size_up, cell_size_down):
    del gt_label

    logits, cidx = pl.pallas_call(
        _tc_argmax_copy,
        grid=(_B // _BB, _NJ),
        in_specs=[pl.BlockSpec((_BB, _CB), lambda i, j: (i, j))],
        out_specs=[
            pl.BlockSpec((_BB, _CB), lambda i, j: (i, j)),
            pl.BlockSpec((_BB,), lambda i, j: (i,)),
        ],
        out_shape=[
            jax.ShapeDtypeStruct((_B, _FD), jnp.float32),
            jax.ShapeDtypeStruct((_B,), jnp.int32),
        ],
        scratch_shapes=[
            pltpu.VMEM((_BB,), jnp.float32),
            pltpu.VMEM((_BB,), jnp.int32),
        ],
        compiler_params=pltpu.CompilerParams(
            dimension_semantics=("parallel", "arbitrary"),
        ),
    )(x)

    f32 = jnp.float32
    sc = pl.kernel(
        _sc_gather_tail,
        mesh=plsc.VectorSubcoreMesh(core_axis_name="c", subcore_axis_name="s"),
        out_type=[
            jax.ShapeDtypeStruct((2 * _B,), f32),
            jax.ShapeDtypeStruct((2 * _B,), f32),
            jax.ShapeDtypeStruct((2 * _B,), f32),
            jax.ShapeDtypeStruct((2 * _B,), f32),
        ],
        scratch_types=(
            [pltpu.VMEM((_RPW,), jnp.int32)] * 5
            + [pltpu.VMEM((_RPW,), f32)] * 8
            + [pltpu.VMEM((2 * _RPW,), f32)] * 4
            + [pltpu.SemaphoreType.DMA]
        ),
    )
    gps_f, inv_f, ctr_f, reg_f = sc(
        x.reshape(-1), cidx,
        cell_center.reshape(-1), cell_size_up.reshape(-1),
        cell_size_down.reshape(-1))

    return (logits,
            gps_f.reshape(2, _B).T,
            inv_f.reshape(2, _B).T,
            ctr_f.reshape(2, _B).T,
            reg_f.reshape(2, _B).T)
